# Initial kernel scaffold; baseline (speedup 1.0000x reference)
import jax
import jax.numpy as jnp
from jax.experimental import pallas as pl


def kernel(inputs, edge_index, W1, b1, W2, b2, W3, b3):
    raise NotImplementedError("write your pallas kernel here")


# speedup vs baseline: 11.7207x; 11.7207x over previous
import functools

import jax
import jax.numpy as jnp
from jax import lax
from jax.experimental import pallas as pl
from jax.experimental.pallas import tpu as pltpu
from jax.experimental.pallas import tpu_sc as plsc

N = 10000
E = 320000
D = 128
H = 128
C = 16
L = 16
NC = 2
NS = 16
NW = NC * NS
NPAD = 10240
EPAD = 327680
EPW = EPAD // NW
CHUNK = 128
NCHUNKS = EPW // CHUNK
RPS = NPAD // NS


def _sc_edge_pass(src, dst, x, w3pad):
    mesh = plsc.VectorSubcoreMesh(
        core_axis_name="c", subcore_axis_name="s", num_cores=NC, num_subcores=NS
    )

    @functools.partial(
        pl.kernel,
        out_type=(
            jax.ShapeDtypeStruct((NC, NPAD, D), jnp.float32),
            jax.ShapeDtypeStruct((NC, NPAD), jnp.float32),
        ),
        mesh=mesh,
        scratch_types=[
            pltpu.VMEM_SHARED((NPAD, D), jnp.float32),
            pltpu.VMEM_SHARED((NPAD,), jnp.float32),
            pltpu.VMEM((CHUNK,), jnp.int32),
            pltpu.VMEM((CHUNK,), jnp.int32),
            pltpu.VMEM((CHUNK, D), jnp.float32),
            pltpu.VMEM((CHUNK,), jnp.float32),
            pltpu.SemaphoreType.DMA,
            pltpu.SemaphoreType.DMA,
        ],
    )
    def k(src_hbm, dst_hbm, x_hbm, w3_hbm, p_out, up_out,
          acc_sh, u_sh, idx_s, idx_d, rows, wvals, sem, usem):
        core = lax.axis_index("c")
        sub = lax.axis_index("s")
        wid = core * NS + sub

        zero16 = jnp.zeros((L,), jnp.float32)

        def zero_wvals(i, carry):
            wvals[pl.ds(i * L, L)] = zero16
            return carry

        lax.fori_loop(0, CHUNK // L, zero_wvals, 0)

        def zero_rows(i, carry):
            for j in range(D // L):
                rows[i, pl.ds(j * L, L)] = zero16
            return carry

        lax.fori_loop(0, CHUNK, zero_rows, 0)

        for t in range(RPS // CHUNK):
            pltpu.sync_copy(rows, acc_sh.at[pl.ds(sub * RPS + t * CHUNK, CHUNK)])
            pltpu.sync_copy(wvals, u_sh.at[pl.ds(sub * RPS + t * CHUNK, CHUNK)])
        plsc.subcore_barrier()

        base = wid * EPW

        def body(cidx, carry):
            off = base + cidx * CHUNK
            pltpu.sync_copy(src_hbm.at[pl.ds(off, CHUNK)], idx_s)
            pltpu.sync_copy(dst_hbm.at[pl.ds(off, CHUNK)], idx_d)
            gx = pltpu.async_copy(x_hbm.at[idx_s], rows, sem)
            gw = pltpu.async_copy(w3_hbm.at[idx_d], wvals, usem)
            gx.wait()
            pltpu.sync_copy(rows, acc_sh.at[idx_d], add=True)
            gw.wait()
            pltpu.sync_copy(wvals, u_sh.at[idx_s], add=True)
            return carry

        lax.fori_loop(0, NCHUNKS, body, 0)
        plsc.subcore_barrier()

        pltpu.sync_copy(
            acc_sh.at[pl.ds(sub * RPS, RPS)],
            p_out.at[core, pl.ds(sub * RPS, RPS)],
        )
        pltpu.sync_copy(
            u_sh.at[pl.ds(sub * RPS, RPS)],
            up_out.at[core, pl.ds(sub * RPS, RPS)],
        )

    return k(src, dst, x, w3pad)


def _tc_tail(p, up, W1, b1, W2, b2, w3pad, b3):
    def body(p_ref, up_ref, w1_ref, b1_ref, w2_ref, b2_ref, w3_ref, b3_ref, o_ref):
        agg = p_ref[0] + p_ref[1]
        h1 = jnp.dot(agg, w1_ref[...], preferred_element_type=jnp.float32)
        h1 = jnp.maximum(h1 + b1_ref[...][None, :], 0.0)
        u = up_ref[0] + up_ref[1]
        v = jnp.dot(u[None, :], h1, preferred_element_type=jnp.float32)
        s = jnp.sum(w3_ref[...])
        logits = jnp.dot(v, w2_ref[...], preferred_element_type=jnp.float32)
        logits = logits + s * b2_ref[...][None, :] + b3_ref[...][None, :]
        o_ref[...] = (1.0 / (1.0 + jnp.exp(-logits))).reshape(C, 1)

    return pl.pallas_call(
        body,
        out_shape=jax.ShapeDtypeStruct((C, 1), jnp.float32),
    )(p, up, W1, b1, W2, b2, w3pad, b3)


@jax.jit
def kernel(inputs, edge_index, W1, b1, W2, b2, W3, b3):
    pad = EPAD - E
    ar = jnp.arange(pad, dtype=jnp.int32)
    src = jnp.concatenate([edge_index[0], ar % N])
    dst = jnp.concatenate([edge_index[1], N + ar % (NPAD - N)])
    w3pad = jnp.concatenate([W3[:, 0], jnp.zeros((NPAD - N,), jnp.float32)])
    p, up = _sc_edge_pass(src, dst, inputs, w3pad)
    return _tc_tail(p, up, W1, b1, W2, b2, w3pad, b3)


# speedup vs baseline: 19.9913x; 1.7056x over previous
import functools

import jax
import jax.numpy as jnp
from jax import lax
from jax.experimental import pallas as pl
from jax.experimental.pallas import tpu as pltpu
from jax.experimental.pallas import tpu_sc as plsc

N = 10000
E = 320000
D = 128
H = 128
C = 16
L = 16
NC = 2
NS = 16
NW = NC * NS
NPAD = 10240
EPAD = 327680
EPW = EPAD // NW
CHUNK = 128
NCHUNKS = EPW // CHUNK
RPS = NPAD // NS


def _sc_edge_pass(src, dst, x, w3pad):
    mesh = plsc.VectorSubcoreMesh(
        core_axis_name="c", subcore_axis_name="s", num_cores=NC, num_subcores=NS
    )

    @functools.partial(
        pl.kernel,
        out_type=(
            jax.ShapeDtypeStruct((NC, NPAD, D), jnp.float32),
            jax.ShapeDtypeStruct((NC, NPAD), jnp.float32),
        ),
        mesh=mesh,
        scratch_types=[
            pltpu.VMEM_SHARED((NPAD, D), jnp.float32),
            pltpu.VMEM_SHARED((NPAD,), jnp.float32),
            pltpu.VMEM((2, CHUNK), jnp.int32),
            pltpu.VMEM((2, CHUNK), jnp.int32),
            pltpu.VMEM((2, CHUNK, D), jnp.float32),
            pltpu.VMEM((2, CHUNK), jnp.float32),
            pltpu.VMEM((RPS,), jnp.float32),
            pltpu.SemaphoreType.DMA,
            pltpu.SemaphoreType.DMA,
            pltpu.SemaphoreType.DMA,
            pltpu.SemaphoreType.DMA,
            pltpu.SemaphoreType.DMA,
            pltpu.SemaphoreType.DMA,
        ],
    )
    def k(src_hbm, dst_hbm, x_hbm, w3_hbm, p_out, up_out,
          acc_sh, u_sh, idx_s, idx_d, rows, wvals, zbuf,
          sem0, sem1, usem0, usem1, isem0, isem1):
        core = lax.axis_index("c")
        sub = lax.axis_index("s")
        wid = core * NS + sub

        zero16 = jnp.zeros((L,), jnp.float32)
        sems = (sem0, sem1)
        usems = (usem0, usem1)
        isems = (isem0, isem1)

        def zero_zbuf(i, carry):
            zbuf[pl.ds(i * L, L)] = zero16
            return carry

        lax.fori_loop(0, RPS // L, zero_zbuf, 0)

        def zero_rows(i, carry):
            for j in range(D // L):
                rows[0, i, pl.ds(j * L, L)] = zero16
            return carry

        lax.fori_loop(0, CHUNK, zero_rows, 0)

        for t in range(RPS // CHUNK):
            pltpu.sync_copy(
                rows.at[0], acc_sh.at[pl.ds(sub * RPS + t * CHUNK, CHUNK)]
            )
        pltpu.sync_copy(zbuf, u_sh.at[pl.ds(sub * RPS, RPS)])

        crow0 = wid * NCHUNKS

        def issue_idx(cidx, b):
            pltpu.async_copy(src_hbm.at[crow0 + cidx], idx_s.at[b], isems[b])
            pltpu.async_copy(dst_hbm.at[crow0 + cidx], idx_d.at[b], isems[b])

        def drain_idx(cidx, b):
            pltpu.make_async_copy(src_hbm.at[crow0 + cidx], idx_s.at[b], isems[b]).wait()
            pltpu.make_async_copy(dst_hbm.at[crow0 + cidx], idx_d.at[b], isems[b]).wait()

        def issue_gather(b):
            pltpu.async_copy(x_hbm.at[idx_s.at[b]], rows.at[b], sems[b])
            pltpu.async_copy(w3_hbm.at[idx_d.at[b]], wvals.at[b], usems[b])

        def drain_gather(b):
            pltpu.make_async_copy(x_hbm.at[idx_s.at[b]], rows.at[b], sems[b]).wait()
            pltpu.make_async_copy(w3_hbm.at[idx_d.at[b]], wvals.at[b], usems[b]).wait()

        def scatter(b):
            pltpu.sync_copy(rows.at[b], acc_sh.at[idx_d.at[b]], add=True)
            pltpu.sync_copy(wvals.at[b], u_sh.at[idx_s.at[b]], add=True)

        issue_idx(0, 0)
        drain_idx(0, 0)
        issue_gather(0)
        issue_idx(1, 1)

        def half_step(c, bg, bi):
            drain_gather(bg)
            drain_idx(c + 1, bi)
            issue_gather(bi)
            scatter(bg)
            issue_idx(c + 2, bg)

        def body(i, carry):
            c = i * 2
            half_step(c, 0, 1)
            half_step(c + 1, 1, 0)
            return carry

        lax.fori_loop(0, NCHUNKS // 2 - 1, body, 0)
        drain_gather(0)
        drain_idx(NCHUNKS - 1, 1)
        issue_gather(1)
        scatter(0)
        drain_gather(1)
        scatter(1)
        plsc.subcore_barrier()

        pltpu.sync_copy(
            acc_sh.at[pl.ds(sub * RPS, RPS)],
            p_out.at[core, pl.ds(sub * RPS, RPS)],
        )
        pltpu.sync_copy(
            u_sh.at[pl.ds(sub * RPS, RPS)],
            up_out.at[core, pl.ds(sub * RPS, RPS)],
        )

    return k(src, dst, x, w3pad)


def _tc_tail(p, up, W1, b1, W2, b2, w3pad, b3):
    def body(p_ref, up_ref, w1_ref, b1_ref, w2_ref, b2_ref, w3_ref, b3_ref, o_ref):
        agg = p_ref[0] + p_ref[1]
        h1 = jnp.dot(agg, w1_ref[...], preferred_element_type=jnp.float32)
        h1 = jnp.maximum(h1 + b1_ref[...][None, :], 0.0)
        u = up_ref[0] + up_ref[1]
        v = jnp.dot(u[None, :N], h1[:N], preferred_element_type=jnp.float32)
        s = jnp.sum(w3_ref[...])
        logits = jnp.dot(v, w2_ref[...], preferred_element_type=jnp.float32)
        logits = logits + s * b2_ref[...][None, :] + b3_ref[...][None, :]
        o_ref[...] = (1.0 / (1.0 + jnp.exp(-logits))).reshape(C, 1)

    return pl.pallas_call(
        body,
        out_shape=jax.ShapeDtypeStruct((C, 1), jnp.float32),
    )(p, up, W1, b1, W2, b2, w3pad, b3)


@jax.jit
def kernel(inputs, edge_index, W1, b1, W2, b2, W3, b3):
    pad = EPAD - E
    ar = jnp.arange(pad, dtype=jnp.int32)
    src = jnp.concatenate([edge_index[0], ar % N]).reshape(EPAD // CHUNK, CHUNK)
    dst = jnp.concatenate([edge_index[1], N + ar % (NPAD - N)]).reshape(
        EPAD // CHUNK, CHUNK
    )
    w3pad = jnp.concatenate([W3[:, 0], jnp.zeros((NPAD - N,), jnp.float32)])
    p, up = _sc_edge_pass(src, dst, inputs, w3pad)
    return _tc_tail(p, up, W1, b1, W2, b2, w3pad, b3)


# speedup vs baseline: 21.4648x; 1.0737x over previous
import functools

import jax
import jax.numpy as jnp
from jax import lax
from jax.experimental import pallas as pl
from jax.experimental.pallas import tpu as pltpu
from jax.experimental.pallas import tpu_sc as plsc

N = 10000
E = 320000
D = 128
H = 128
C = 16
L = 16
NC = 2
NS = 16
NW = NC * NS
NPAD = 10240
EPAD = 327680
EPW = EPAD // NW
CHUNK = 80
NCHUNKS = EPW // CHUNK
NBUF = 4
RPS = NPAD // NS


def _sc_edge_pass(src, dst, x, w3pad):
    mesh = plsc.VectorSubcoreMesh(
        core_axis_name="c", subcore_axis_name="s", num_cores=NC, num_subcores=NS
    )

    @functools.partial(
        pl.kernel,
        out_type=(
            jax.ShapeDtypeStruct((NC, NPAD, D), jnp.float32),
            jax.ShapeDtypeStruct((NC, NPAD), jnp.float32),
        ),
        mesh=mesh,
        scratch_types=[
            pltpu.VMEM_SHARED((NPAD, D), jnp.float32),
            pltpu.VMEM_SHARED((NPAD,), jnp.float32),
            pltpu.VMEM((NBUF, CHUNK), jnp.int32),
            pltpu.VMEM((NBUF, CHUNK), jnp.int32),
            pltpu.VMEM((NBUF, CHUNK), jnp.int32),
            pltpu.VMEM((NBUF, CHUNK), jnp.int32),
            pltpu.VMEM((NBUF, CHUNK, D), jnp.float32),
            pltpu.VMEM((NBUF, CHUNK), jnp.float32),
            pltpu.VMEM((RPS,), jnp.float32),
        ] + [pltpu.SemaphoreType.DMA] * (5 * NBUF),
    )
    def k(src_hbm, dst_hbm, x_hbm, w3_hbm, p_out, up_out,
          acc_sh, u_sh, idx_s, idx_d, sidx_s, sidx_d, rows, wvals, zbuf,
          *sems):
        core = lax.axis_index("c")
        sub = lax.axis_index("s")
        wid = core * NS + sub

        isems = sems[0:NBUF]
        gsems = sems[NBUF:2 * NBUF]
        wsems = sems[2 * NBUF:3 * NBUF]
        rsems = sems[3 * NBUF:4 * NBUF]
        usems = sems[4 * NBUF:5 * NBUF]

        zero16 = jnp.zeros((L,), jnp.float32)

        def zero_zbuf(i, carry):
            zbuf[pl.ds(i * L, L)] = zero16
            return carry

        lax.fori_loop(0, RPS // L, zero_zbuf, 0)

        def zero_rows(i, carry):
            for j in range(D // L):
                rows[0, i, pl.ds(j * L, L)] = zero16
            return carry

        lax.fori_loop(0, CHUNK, zero_rows, 0)

        for t in range(RPS // CHUNK):
            pltpu.sync_copy(
                rows.at[0], acc_sh.at[pl.ds(sub * RPS + t * CHUNK, CHUNK)]
            )
        pltpu.sync_copy(zbuf, u_sh.at[pl.ds(sub * RPS, RPS)])

        crow0 = wid * NCHUNKS

        def issue_idx(c, b):
            pltpu.async_copy(src_hbm.at[crow0 + c], idx_s.at[b], isems[b])
            pltpu.async_copy(dst_hbm.at[crow0 + c], idx_d.at[b], isems[b])

        def drain_idx(c, b):
            pltpu.make_async_copy(src_hbm.at[crow0 + c], idx_s.at[b], isems[b]).wait()
            pltpu.make_async_copy(dst_hbm.at[crow0 + c], idx_d.at[b], isems[b]).wait()

        def issue_gather(b):
            pltpu.async_copy(x_hbm.at[idx_s.at[b]], rows.at[b], gsems[b])
            pltpu.async_copy(w3_hbm.at[idx_d.at[b]], wvals.at[b], wsems[b])

        def drain_gather(b):
            pltpu.make_async_copy(x_hbm.at[idx_s.at[b]], rows.at[b], gsems[b]).wait()
            pltpu.make_async_copy(w3_hbm.at[idx_d.at[b]], wvals.at[b], wsems[b]).wait()

        def copy_sidx(b):
            for j in range(CHUNK // L):
                sidx_s[b, pl.ds(j * L, L)] = idx_s[b, pl.ds(j * L, L)]
                sidx_d[b, pl.ds(j * L, L)] = idx_d[b, pl.ds(j * L, L)]

        def issue_scatter(b):
            pltpu.async_copy(rows.at[b], acc_sh.at[sidx_d.at[b]], rsems[b], add=True)
            pltpu.async_copy(wvals.at[b], u_sh.at[sidx_s.at[b]], usems[b], add=True)

        def drain_scatter(b):
            pltpu.make_async_copy(rows.at[b], acc_sh.at[sidx_d.at[b]], rsems[b]).wait()
            pltpu.make_async_copy(wvals.at[b], u_sh.at[sidx_s.at[b]], usems[b]).wait()


        issue_idx(0, 0)
        issue_idx(1, 1)
        issue_idx(2, 2)
        drain_idx(0, 0)
        issue_gather(0)
        issue_idx(3, 3)
        drain_idx(1, 1)
        issue_gather(1)
        drain_gather(0)
        copy_sidx(0)
        issue_scatter(0)
        issue_idx(4, 0)
        drain_idx(2, 2)
        issue_gather(2)
        drain_gather(1)
        copy_sidx(1)
        issue_scatter(1)
        issue_idx(5, 1)
        drain_idx(3, 3)
        issue_gather(3)
        drain_gather(2)
        copy_sidx(2)
        issue_scatter(2)

        def step(i, b):
            drain_scatter(b)
            issue_idx(i + 2, (b + 2) % NBUF)
            drain_idx(i, b)
            issue_gather(b)
            drain_gather((b - 1) % NBUF)
            copy_sidx((b - 1) % NBUF)
            issue_scatter((b - 1) % NBUF)

        def body(g, carry):
            i0 = g * NBUF
            step(i0 + 0, 0)
            step(i0 + 1, 1)
            step(i0 + 2, 2)
            step(i0 + 3, 3)
            return carry

        lax.fori_loop(1, NCHUNKS // NBUF - 1, body, 0)

        for i in (NCHUNKS - 4, NCHUNKS - 3):
            b = i % NBUF
            drain_scatter(b)
            issue_idx(i + 2, (i + 2) % NBUF)
            drain_idx(i, b)
            issue_gather(b)
            drain_gather((i - 1) % NBUF)
            copy_sidx((i - 1) % NBUF)
            issue_scatter((i - 1) % NBUF)
        for i in (NCHUNKS - 2, NCHUNKS - 1):
            b = i % NBUF
            drain_scatter(b)
            drain_idx(i, b)
            issue_gather(b)
            drain_gather((i - 1) % NBUF)
            copy_sidx((i - 1) % NBUF)
            issue_scatter((i - 1) % NBUF)
        bl = (NCHUNKS - 1) % NBUF
        drain_gather(bl)
        copy_sidx(bl)
        issue_scatter(bl)
        for i in range(NCHUNKS - 4, NCHUNKS):
            drain_scatter(i % NBUF)
        plsc.subcore_barrier()

        pltpu.sync_copy(
            acc_sh.at[pl.ds(sub * RPS, RPS)],
            p_out.at[core, pl.ds(sub * RPS, RPS)],
        )
        pltpu.sync_copy(
            u_sh.at[pl.ds(sub * RPS, RPS)],
            up_out.at[core, pl.ds(sub * RPS, RPS)],
        )

    return k(src, dst, x, w3pad)


def _tc_tail(p, up, W1, b1, W2, b2, w3pad, b3):
    def body(p_ref, up_ref, w1_ref, b1_ref, w2_ref, b2_ref, w3_ref, b3_ref, o_ref):
        agg = p_ref[0] + p_ref[1]
        h1 = jnp.dot(agg, w1_ref[...], preferred_element_type=jnp.float32)
        h1 = jnp.maximum(h1 + b1_ref[...][None, :], 0.0)
        u = up_ref[0] + up_ref[1]
        v = jnp.dot(u[None, :N], h1[:N], preferred_element_type=jnp.float32)
        s = jnp.sum(w3_ref[...])
        logits = jnp.dot(v, w2_ref[...], preferred_element_type=jnp.float32)
        logits = logits + s * b2_ref[...][None, :] + b3_ref[...][None, :]
        o_ref[...] = (1.0 / (1.0 + jnp.exp(-logits))).reshape(C, 1)

    return pl.pallas_call(
        body,
        out_shape=jax.ShapeDtypeStruct((C, 1), jnp.float32),
    )(p, up, W1, b1, W2, b2, w3pad, b3)


@jax.jit
def kernel(inputs, edge_index, W1, b1, W2, b2, W3, b3):
    pad = EPAD - E
    ar = jnp.arange(pad, dtype=jnp.int32)
    src = jnp.concatenate([edge_index[0], ar % N]).reshape(EPAD // CHUNK, CHUNK)
    dst = jnp.concatenate([edge_index[1], N + ar % (NPAD - N)]).reshape(
        EPAD // CHUNK, CHUNK
    )
    w3pad = jnp.concatenate([W3[:, 0], jnp.zeros((NPAD - N,), jnp.float32)])
    p, up = _sc_edge_pass(src, dst, inputs, w3pad)
    return _tc_tail(p, up, W1, b1, W2, b2, w3pad, b3)


# speedup vs baseline: 21.8870x; 1.0197x over previous
import functools

import jax
import jax.numpy as jnp
from jax import lax
from jax.experimental import pallas as pl
from jax.experimental.pallas import tpu as pltpu
from jax.experimental.pallas import tpu_sc as plsc

N = 10000
E = 320000
D = 128
H = 128
C = 16
L = 16
NC = 2
NS = 16
NW = NC * NS
CHUNK = 80
NCHUNKS = E // CHUNK // NW
NBUF = 4
RPS = 640
RPS_LAST = N - 15 * RPS
UPAD = NS * RPS


def _sc_edge_pass(src, dst, x, w3pad):
    mesh = plsc.VectorSubcoreMesh(
        core_axis_name="c", subcore_axis_name="s", num_cores=NC, num_subcores=NS
    )

    @functools.partial(
        pl.kernel,
        out_type=(
            jax.ShapeDtypeStruct((NC, N, D), jnp.float32),
            jax.ShapeDtypeStruct((NC, UPAD), jnp.float32),
        ),
        mesh=mesh,
        scratch_types=[
            pltpu.VMEM_SHARED((N, D), jnp.float32),
            pltpu.VMEM_SHARED((UPAD,), jnp.float32),
            pltpu.VMEM((NBUF, CHUNK), jnp.int32),
            pltpu.VMEM((NBUF, CHUNK), jnp.int32),
            pltpu.VMEM((NBUF, CHUNK), jnp.int32),
            pltpu.VMEM((NBUF, CHUNK), jnp.int32),
            pltpu.VMEM((NBUF, CHUNK, D), jnp.float32),
            pltpu.VMEM((NBUF, CHUNK), jnp.float32),
            pltpu.VMEM((RPS,), jnp.float32),
        ] + [pltpu.SemaphoreType.DMA] * (5 * NBUF),
    )
    def k(src_hbm, dst_hbm, x_hbm, w3_hbm, p_out, up_out,
          acc_sh, u_sh, idx_s, idx_d, sidx_s, sidx_d, rows, wvals, zbuf,
          *sems):
        core = lax.axis_index("c")
        sub = lax.axis_index("s")
        wid = core * NS + sub

        isems = sems[0:NBUF]
        gsems = sems[NBUF:2 * NBUF]
        wsems = sems[2 * NBUF:3 * NBUF]
        rsems = sems[3 * NBUF:4 * NBUF]
        usems = sems[4 * NBUF:5 * NBUF]

        zero16 = jnp.zeros((L,), jnp.float32)

        def zero_zbuf(i, carry):
            zbuf[pl.ds(i * L, L)] = zero16
            return carry

        lax.fori_loop(0, RPS // L, zero_zbuf, 0)

        def zero_rows(i, carry):
            for j in range(D // L):
                rows[0, i, pl.ds(j * L, L)] = zero16
            return carry

        lax.fori_loop(0, CHUNK, zero_rows, 0)

        pltpu.sync_copy(zbuf, u_sh.at[pl.ds(sub * RPS, RPS)])

        @pl.when(sub < NS - 1)
        def _zero_main():
            for t in range(RPS // CHUNK):
                pltpu.sync_copy(
                    rows.at[0], acc_sh.at[pl.ds(sub * RPS + t * CHUNK, CHUNK)]
                )

        @pl.when(sub == NS - 1)
        def _zero_last():
            for t in range(RPS_LAST // CHUNK):
                pltpu.sync_copy(
                    rows.at[0],
                    acc_sh.at[pl.ds((NS - 1) * RPS + t * CHUNK, CHUNK)],
                )

        crow0 = wid * NCHUNKS

        def issue_idx(c, b):
            pltpu.async_copy(src_hbm.at[crow0 + c], idx_s.at[b], isems[b])
            pltpu.async_copy(dst_hbm.at[crow0 + c], idx_d.at[b], isems[b])

        def drain_idx(c, b):
            pltpu.make_async_copy(src_hbm.at[crow0 + c], idx_s.at[b], isems[b]).wait()
            pltpu.make_async_copy(dst_hbm.at[crow0 + c], idx_d.at[b], isems[b]).wait()

        def issue_gather(b):
            pltpu.async_copy(x_hbm.at[idx_s.at[b]], rows.at[b], gsems[b])
            pltpu.async_copy(w3_hbm.at[idx_d.at[b]], wvals.at[b], wsems[b])

        def drain_gather(b):
            pltpu.make_async_copy(x_hbm.at[idx_s.at[b]], rows.at[b], gsems[b]).wait()
            pltpu.make_async_copy(w3_hbm.at[idx_d.at[b]], wvals.at[b], wsems[b]).wait()

        def copy_sidx(b):
            for j in range(CHUNK // L):
                sidx_s[b, pl.ds(j * L, L)] = idx_s[b, pl.ds(j * L, L)]
                sidx_d[b, pl.ds(j * L, L)] = idx_d[b, pl.ds(j * L, L)]

        def issue_scatter(b):
            pltpu.async_copy(rows.at[b], acc_sh.at[sidx_d.at[b]], rsems[b], add=True)
            pltpu.async_copy(wvals.at[b], u_sh.at[sidx_s.at[b]], usems[b], add=True)

        def drain_scatter(b):
            pltpu.make_async_copy(rows.at[b], acc_sh.at[sidx_d.at[b]], rsems[b]).wait()
            pltpu.make_async_copy(wvals.at[b], u_sh.at[sidx_s.at[b]], usems[b]).wait()


        issue_idx(0, 0)
        issue_idx(1, 1)
        issue_idx(2, 2)
        drain_idx(0, 0)
        issue_gather(0)
        issue_idx(3, 3)
        drain_idx(1, 1)
        issue_gather(1)
        drain_gather(0)
        copy_sidx(0)
        issue_scatter(0)
        issue_idx(4, 0)
        drain_idx(2, 2)
        issue_gather(2)
        drain_gather(1)
        copy_sidx(1)
        issue_scatter(1)
        issue_idx(5, 1)
        drain_idx(3, 3)
        issue_gather(3)
        drain_gather(2)
        copy_sidx(2)
        issue_scatter(2)

        def step(i, b):
            drain_scatter(b)
            issue_idx(i + 2, (b + 2) % NBUF)
            drain_idx(i, b)
            issue_gather(b)
            drain_gather((b - 1) % NBUF)
            copy_sidx((b - 1) % NBUF)
            issue_scatter((b - 1) % NBUF)

        def body(g, carry):
            i0 = g * NBUF
            step(i0 + 0, 0)
            step(i0 + 1, 1)
            step(i0 + 2, 2)
            step(i0 + 3, 3)
            return carry

        lax.fori_loop(1, (NCHUNKS - 1) // NBUF - 1, body, 0)

        for i in range(NCHUNKS - 5, NCHUNKS - 2):
            b = i % NBUF
            drain_scatter(b)
            issue_idx(i + 2, (b + 2) % NBUF)
            drain_idx(i, b)
            issue_gather(b)
            drain_gather((b - 1) % NBUF)
            copy_sidx((b - 1) % NBUF)
            issue_scatter((b - 1) % NBUF)
        for i in (NCHUNKS - 2, NCHUNKS - 1):
            b = i % NBUF
            drain_scatter(b)
            drain_idx(i, b)
            issue_gather(b)
            drain_gather((b - 1) % NBUF)
            copy_sidx((b - 1) % NBUF)
            issue_scatter((b - 1) % NBUF)
        bl = (NCHUNKS - 1) % NBUF
        drain_gather(bl)
        copy_sidx(bl)
        issue_scatter(bl)
        for i in range(NCHUNKS - 4, NCHUNKS):
            drain_scatter(i % NBUF)
        plsc.subcore_barrier()

        pltpu.sync_copy(
            u_sh.at[pl.ds(sub * RPS, RPS)],
            up_out.at[core, pl.ds(sub * RPS, RPS)],
        )

        @pl.when(sub < NS - 1)
        def _pub_main():
            pltpu.sync_copy(
                acc_sh.at[pl.ds(sub * RPS, RPS)],
                p_out.at[core, pl.ds(sub * RPS, RPS)],
            )

        @pl.when(sub == NS - 1)
        def _pub_last():
            pltpu.sync_copy(
                acc_sh.at[pl.ds((NS - 1) * RPS, RPS_LAST)],
                p_out.at[core, pl.ds((NS - 1) * RPS, RPS_LAST)],
            )

    return k(src, dst, x, w3pad)


def _tc_tail(p, up, W1, b1, W2, b2, w3pad, b3):
    def body(p_ref, up_ref, w1_ref, b1_ref, w2_ref, b2_ref, w3_ref, b3_ref, o_ref):
        agg = p_ref[0] + p_ref[1]
        h1 = jnp.dot(agg, w1_ref[...], preferred_element_type=jnp.float32)
        h1 = jnp.maximum(h1 + b1_ref[...][None, :], 0.0)
        u = (up_ref[0] + up_ref[1])[:N]
        v = jnp.dot(u[None, :], h1, preferred_element_type=jnp.float32)
        s = jnp.sum(w3_ref[...])
        logits = jnp.dot(v, w2_ref[...], preferred_element_type=jnp.float32)
        logits = logits + s * b2_ref[...][None, :] + b3_ref[...][None, :]
        o_ref[...] = (1.0 / (1.0 + jnp.exp(-logits))).reshape(C, 1)

    return pl.pallas_call(
        body,
        out_shape=jax.ShapeDtypeStruct((C, 1), jnp.float32),
    )(p, up, W1, b1, W2, b2, w3pad, b3)


@jax.jit
def kernel(inputs, edge_index, W1, b1, W2, b2, W3, b3):
    src = edge_index[0].reshape(E // CHUNK, CHUNK)
    dst = edge_index[1].reshape(E // CHUNK, CHUNK)
    w3flat = W3[:, 0]
    p, up = _sc_edge_pass(src, dst, inputs, w3flat)
    return _tc_tail(p, up, W1, b1, W2, b2, w3flat, b3)


# speedup vs baseline: 21.9458x; 1.0027x over previous
import functools

import jax
import jax.numpy as jnp
from jax import lax
from jax.experimental import pallas as pl
from jax.experimental.pallas import tpu as pltpu
from jax.experimental.pallas import tpu_sc as plsc

N = 10000
E = 320000
D = 128
H = 128
C = 16
L = 16
NC = 2
NS = 16
NW = NC * NS
CHUNK = 80
NCHUNKS = E // CHUNK // NW
NBUF = 4
RPS = 640
RPS_LAST = N - 15 * RPS
UPAD = NS * RPS


def _sc_edge_pass(src, dst, x, w3pad):
    mesh = plsc.VectorSubcoreMesh(
        core_axis_name="c", subcore_axis_name="s", num_cores=NC, num_subcores=NS
    )

    @functools.partial(
        pl.kernel,
        out_type=(
            jax.ShapeDtypeStruct((NC, N, D), jnp.float32),
            jax.ShapeDtypeStruct((NC, UPAD), jnp.float32),
        ),
        mesh=mesh,
        scratch_types=[
            pltpu.VMEM_SHARED((N, D), jnp.float32),
            pltpu.VMEM_SHARED((UPAD,), jnp.float32),
            pltpu.VMEM((NBUF, CHUNK), jnp.int32),
            pltpu.VMEM((NBUF, CHUNK), jnp.int32),
            pltpu.VMEM((NBUF, CHUNK), jnp.int32),
            pltpu.VMEM((NBUF, CHUNK), jnp.int32),
            pltpu.VMEM((NBUF, CHUNK, D), jnp.float32),
            pltpu.VMEM((NBUF, CHUNK), jnp.float32),
            pltpu.VMEM((RPS,), jnp.float32),
        ] + [pltpu.SemaphoreType.DMA] * (5 * NBUF + 1),
    )
    def k(src_hbm, dst_hbm, x_hbm, w3_hbm, p_out, up_out,
          acc_sh, u_sh, idx_s, idx_d, sidx_s, sidx_d, rows, wvals, zbuf,
          *sems):
        core = lax.axis_index("c")
        sub = lax.axis_index("s")
        wid = core * NS + sub

        isems = sems[0:NBUF]
        gsems = sems[NBUF:2 * NBUF]
        wsems = sems[2 * NBUF:3 * NBUF]
        rsems = sems[3 * NBUF:4 * NBUF]
        usems = sems[4 * NBUF:5 * NBUF]
        zsem = sems[5 * NBUF]

        zero16 = jnp.zeros((L,), jnp.float32)

        crow0 = wid * NCHUNKS

        def issue_idx(c, b):
            pltpu.async_copy(src_hbm.at[crow0 + c], idx_s.at[b], isems[b])
            pltpu.async_copy(dst_hbm.at[crow0 + c], idx_d.at[b], isems[b])

        def drain_idx(c, b):
            pltpu.make_async_copy(src_hbm.at[crow0 + c], idx_s.at[b], isems[b]).wait()
            pltpu.make_async_copy(dst_hbm.at[crow0 + c], idx_d.at[b], isems[b]).wait()

        def issue_gather(b):
            pltpu.async_copy(x_hbm.at[idx_s.at[b]], rows.at[b], gsems[b])
            pltpu.async_copy(w3_hbm.at[idx_d.at[b]], wvals.at[b], wsems[b])

        def drain_gather(b):
            pltpu.make_async_copy(x_hbm.at[idx_s.at[b]], rows.at[b], gsems[b]).wait()
            pltpu.make_async_copy(w3_hbm.at[idx_d.at[b]], wvals.at[b], wsems[b]).wait()

        def copy_sidx(b):
            for j in range(CHUNK // L):
                sidx_s[b, pl.ds(j * L, L)] = idx_s[b, pl.ds(j * L, L)]
                sidx_d[b, pl.ds(j * L, L)] = idx_d[b, pl.ds(j * L, L)]

        def issue_scatter(b):
            pltpu.async_copy(rows.at[b], acc_sh.at[sidx_d.at[b]], rsems[b], add=True)
            pltpu.async_copy(wvals.at[b], u_sh.at[sidx_s.at[b]], usems[b], add=True)

        def drain_scatter(b):
            pltpu.make_async_copy(rows.at[b], acc_sh.at[sidx_d.at[b]], rsems[b]).wait()
            pltpu.make_async_copy(wvals.at[b], u_sh.at[sidx_s.at[b]], usems[b]).wait()


        issue_idx(0, 0)
        issue_idx(1, 1)
        issue_idx(2, 2)

        def zero_zbuf(i, carry):
            zbuf[pl.ds(i * L, L)] = zero16
            return carry

        lax.fori_loop(0, RPS // L, zero_zbuf, 0)

        def zero_rows(i, carry):
            for j in range(D // L):
                rows[0, i, pl.ds(j * L, L)] = zero16
            return carry

        lax.fori_loop(0, CHUNK, zero_rows, 0)

        pltpu.async_copy(zbuf, u_sh.at[pl.ds(sub * RPS, RPS)], zsem)

        @pl.when(sub < NS - 1)
        def _zero_main():
            for t in range(RPS // CHUNK):
                pltpu.async_copy(
                    rows.at[0], acc_sh.at[pl.ds(sub * RPS + t * CHUNK, CHUNK)],
                    zsem,
                )

        @pl.when(sub == NS - 1)
        def _zero_last():
            for t in range(RPS_LAST // CHUNK):
                pltpu.async_copy(
                    rows.at[0],
                    acc_sh.at[pl.ds((NS - 1) * RPS + t * CHUNK, CHUNK)],
                    zsem,
                )

        pltpu.make_async_copy(zbuf, u_sh.at[pl.ds(sub * RPS, RPS)], zsem).wait()

        @pl.when(sub < NS - 1)
        def _zero_main_wait():
            for t in range(RPS // CHUNK):
                pltpu.make_async_copy(
                    rows.at[0], acc_sh.at[pl.ds(sub * RPS + t * CHUNK, CHUNK)],
                    zsem,
                ).wait()

        @pl.when(sub == NS - 1)
        def _zero_last_wait():
            for t in range(RPS_LAST // CHUNK):
                pltpu.make_async_copy(
                    rows.at[0],
                    acc_sh.at[pl.ds((NS - 1) * RPS + t * CHUNK, CHUNK)],
                    zsem,
                ).wait()

        plsc.subcore_barrier()

        drain_idx(0, 0)
        issue_gather(0)
        issue_idx(3, 3)
        drain_idx(1, 1)
        issue_gather(1)
        drain_gather(0)
        copy_sidx(0)
        issue_scatter(0)
        issue_idx(4, 0)
        drain_idx(2, 2)
        issue_gather(2)
        drain_gather(1)
        copy_sidx(1)
        issue_scatter(1)
        issue_idx(5, 1)
        drain_idx(3, 3)
        issue_gather(3)
        drain_gather(2)
        copy_sidx(2)
        issue_scatter(2)

        def step(i, b):
            drain_scatter(b)
            issue_idx(i + 2, (b + 2) % NBUF)
            drain_idx(i, b)
            issue_gather(b)
            drain_gather((b - 1) % NBUF)
            copy_sidx((b - 1) % NBUF)
            issue_scatter((b - 1) % NBUF)

        def body(g, carry):
            i0 = g * NBUF
            step(i0 + 0, 0)
            step(i0 + 1, 1)
            step(i0 + 2, 2)
            step(i0 + 3, 3)
            return carry

        lax.fori_loop(1, (NCHUNKS - 1) // NBUF - 1, body, 0)

        for i in range(NCHUNKS - 5, NCHUNKS - 2):
            b = i % NBUF
            drain_scatter(b)
            issue_idx(i + 2, (b + 2) % NBUF)
            drain_idx(i, b)
            issue_gather(b)
            drain_gather((b - 1) % NBUF)
            copy_sidx((b - 1) % NBUF)
            issue_scatter((b - 1) % NBUF)
        for i in (NCHUNKS - 2, NCHUNKS - 1):
            b = i % NBUF
            drain_scatter(b)
            drain_idx(i, b)
            issue_gather(b)
            drain_gather((b - 1) % NBUF)
            copy_sidx((b - 1) % NBUF)
            issue_scatter((b - 1) % NBUF)
        bl = (NCHUNKS - 1) % NBUF
        drain_gather(bl)
        copy_sidx(bl)
        issue_scatter(bl)
        for i in range(NCHUNKS - 4, NCHUNKS):
            drain_scatter(i % NBUF)
        plsc.subcore_barrier()

        pltpu.sync_copy(
            u_sh.at[pl.ds(sub * RPS, RPS)],
            up_out.at[core, pl.ds(sub * RPS, RPS)],
        )

        @pl.when(sub < NS - 1)
        def _pub_main():
            pltpu.sync_copy(
                acc_sh.at[pl.ds(sub * RPS, RPS)],
                p_out.at[core, pl.ds(sub * RPS, RPS)],
            )

        @pl.when(sub == NS - 1)
        def _pub_last():
            pltpu.sync_copy(
                acc_sh.at[pl.ds((NS - 1) * RPS, RPS_LAST)],
                p_out.at[core, pl.ds((NS - 1) * RPS, RPS_LAST)],
            )

    return k(src, dst, x, w3pad)


def _tc_tail(p, up, W1, b1, W2, b2, w3pad, b3):
    def body(p_ref, up_ref, w1_ref, b1_ref, w2_ref, b2_ref, w3_ref, b3_ref, o_ref):
        agg = p_ref[0] + p_ref[1]
        h1 = jnp.dot(agg, w1_ref[...], preferred_element_type=jnp.float32)
        h1 = jnp.maximum(h1 + b1_ref[...][None, :], 0.0)
        u = (up_ref[0] + up_ref[1])[:N]
        v = jnp.dot(u[None, :], h1, preferred_element_type=jnp.float32)
        s = jnp.sum(w3_ref[...])
        logits = jnp.dot(v, w2_ref[...], preferred_element_type=jnp.float32)
        logits = logits + s * b2_ref[...][None, :] + b3_ref[...][None, :]
        o_ref[...] = (1.0 / (1.0 + jnp.exp(-logits))).reshape(C, 1)

    return pl.pallas_call(
        body,
        out_shape=jax.ShapeDtypeStruct((C, 1), jnp.float32),
    )(p, up, W1, b1, W2, b2, w3pad, b3)


@jax.jit
def kernel(inputs, edge_index, W1, b1, W2, b2, W3, b3):
    src = edge_index[0].reshape(E // CHUNK, CHUNK)
    dst = edge_index[1].reshape(E // CHUNK, CHUNK)
    w3flat = W3[:, 0]
    p, up = _sc_edge_pass(src, dst, inputs, w3flat)
    return _tc_tail(p, up, W1, b1, W2, b2, w3flat, b3)


# speedup vs baseline: 22.1363x; 1.0087x over previous
import functools

import jax
import jax.numpy as jnp
from jax import lax
from jax.experimental import pallas as pl
from jax.experimental.pallas import tpu as pltpu
from jax.experimental.pallas import tpu_sc as plsc

N = 10000
E = 320000
D = 128
H = 128
C = 16
L = 16
NC = 2
NS = 16
NW = NC * NS
CHUNK = 80
NCHUNKS = E // CHUNK // NW
NBUF = 4
RPS = 640
RPS_LAST = N - 15 * RPS
UPAD = NS * RPS


def _sc_edge_pass(src, dst, x, w3pad):
    mesh = plsc.VectorSubcoreMesh(
        core_axis_name="c", subcore_axis_name="s", num_cores=NC, num_subcores=NS
    )

    @functools.partial(
        pl.kernel,
        out_type=(
            jax.ShapeDtypeStruct((NC, N, D), jnp.float32),
            jax.ShapeDtypeStruct((NC, UPAD), jnp.float32),
        ),
        mesh=mesh,
        scratch_types=[
            pltpu.VMEM_SHARED((N, D), jnp.float32),
            pltpu.VMEM_SHARED((UPAD,), jnp.float32),
            pltpu.VMEM((NBUF, CHUNK), jnp.int32),
            pltpu.VMEM((NBUF, CHUNK), jnp.int32),
            pltpu.VMEM((NBUF, CHUNK), jnp.int32),
            pltpu.VMEM((NBUF, CHUNK), jnp.int32),
            pltpu.VMEM((NBUF, CHUNK, D), jnp.float32),
            pltpu.VMEM((NBUF, CHUNK), jnp.float32),
            pltpu.VMEM((RPS,), jnp.float32),
        ] + [pltpu.SemaphoreType.DMA] * (5 * NBUF + 1),
    )
    def k(src_hbm, dst_hbm, x_hbm, w3_hbm, p_out, up_out,
          acc_sh, u_sh, idx_s, idx_d, sidx_s, sidx_d, rows, wvals, zbuf,
          *sems):
        core = lax.axis_index("c")
        sub = lax.axis_index("s")
        wid = core * NS + sub

        isems = sems[0:NBUF]
        gsems = sems[NBUF:2 * NBUF]
        wsems = sems[2 * NBUF:3 * NBUF]
        rsems = sems[3 * NBUF:4 * NBUF]
        usems = sems[4 * NBUF:5 * NBUF]
        zsem = sems[5 * NBUF]

        zero16 = jnp.zeros((L,), jnp.float32)

        crow0 = wid * NCHUNKS

        def issue_idx(c, b):
            pltpu.async_copy(src_hbm.at[crow0 + c], idx_s.at[b], isems[b])
            pltpu.async_copy(dst_hbm.at[crow0 + c], idx_d.at[b], isems[b])

        def drain_idx(c, b):
            pltpu.make_async_copy(src_hbm.at[crow0 + c], idx_s.at[b], isems[b]).wait()
            pltpu.make_async_copy(dst_hbm.at[crow0 + c], idx_d.at[b], isems[b]).wait()

        def issue_gather(b):
            pltpu.async_copy(x_hbm.at[idx_s.at[b]], rows.at[b], gsems[b])
            pltpu.async_copy(w3_hbm.at[idx_d.at[b]], wvals.at[b], wsems[b])

        def drain_gather(b):
            pltpu.make_async_copy(x_hbm.at[idx_s.at[b]], rows.at[b], gsems[b]).wait()
            pltpu.make_async_copy(w3_hbm.at[idx_d.at[b]], wvals.at[b], wsems[b]).wait()

        def copy_sidx(b):
            for j in range(CHUNK // L):
                sidx_s[b, pl.ds(j * L, L)] = idx_s[b, pl.ds(j * L, L)]
                sidx_d[b, pl.ds(j * L, L)] = idx_d[b, pl.ds(j * L, L)]

        def issue_scatter(b):
            pltpu.async_copy(rows.at[b], acc_sh.at[sidx_d.at[b]], rsems[b], add=True)
            pltpu.async_copy(wvals.at[b], u_sh.at[sidx_s.at[b]], usems[b], add=True)

        def drain_scatter(b):
            pltpu.make_async_copy(rows.at[b], acc_sh.at[sidx_d.at[b]], rsems[b]).wait()
            pltpu.make_async_copy(wvals.at[b], u_sh.at[sidx_s.at[b]], usems[b]).wait()


        issue_idx(0, 0)
        issue_idx(1, 1)
        issue_idx(2, 2)

        def zero_zbuf(i, carry):
            zbuf[pl.ds(i * L, L)] = zero16
            return carry

        lax.fori_loop(0, RPS // L, zero_zbuf, 0)

        def zero_rows(i, carry):
            for j in range(D // L):
                rows[3, i, pl.ds(j * L, L)] = zero16
            return carry

        lax.fori_loop(0, CHUNK, zero_rows, 0)

        pltpu.async_copy(zbuf, u_sh.at[pl.ds(sub * RPS, RPS)], zsem)

        @pl.when(sub < NS - 1)
        def _zero_main():
            for t in range(RPS // CHUNK):
                pltpu.async_copy(
                    rows.at[3], acc_sh.at[pl.ds(sub * RPS + t * CHUNK, CHUNK)],
                    zsem,
                )

        @pl.when(sub == NS - 1)
        def _zero_last():
            for t in range(RPS_LAST // CHUNK):
                pltpu.async_copy(
                    rows.at[3],
                    acc_sh.at[pl.ds((NS - 1) * RPS + t * CHUNK, CHUNK)],
                    zsem,
                )

        drain_idx(0, 0)
        issue_gather(0)
        issue_idx(3, 3)
        drain_idx(1, 1)
        issue_gather(1)
        drain_gather(0)
        copy_sidx(0)

        pltpu.make_async_copy(zbuf, u_sh.at[pl.ds(sub * RPS, RPS)], zsem).wait()

        @pl.when(sub < NS - 1)
        def _zero_main_wait():
            for t in range(RPS // CHUNK):
                pltpu.make_async_copy(
                    rows.at[3], acc_sh.at[pl.ds(sub * RPS + t * CHUNK, CHUNK)],
                    zsem,
                ).wait()

        @pl.when(sub == NS - 1)
        def _zero_last_wait():
            for t in range(RPS_LAST // CHUNK):
                pltpu.make_async_copy(
                    rows.at[3],
                    acc_sh.at[pl.ds((NS - 1) * RPS + t * CHUNK, CHUNK)],
                    zsem,
                ).wait()

        plsc.subcore_barrier()

        issue_scatter(0)
        issue_idx(4, 0)
        drain_idx(2, 2)
        issue_gather(2)
        drain_gather(1)
        copy_sidx(1)
        issue_scatter(1)
        issue_idx(5, 1)
        drain_idx(3, 3)
        issue_gather(3)
        drain_gather(2)
        copy_sidx(2)
        issue_scatter(2)

        def step(i, b):
            drain_scatter(b)
            issue_idx(i + 2, (b + 2) % NBUF)
            drain_idx(i, b)
            issue_gather(b)
            drain_gather((b - 1) % NBUF)
            copy_sidx((b - 1) % NBUF)
            issue_scatter((b - 1) % NBUF)

        def body(g, carry):
            i0 = g * NBUF
            step(i0 + 0, 0)
            step(i0 + 1, 1)
            step(i0 + 2, 2)
            step(i0 + 3, 3)
            return carry

        lax.fori_loop(1, (NCHUNKS - 1) // NBUF - 1, body, 0)

        for i in range(NCHUNKS - 5, NCHUNKS - 2):
            b = i % NBUF
            drain_scatter(b)
            issue_idx(i + 2, (b + 2) % NBUF)
            drain_idx(i, b)
            issue_gather(b)
            drain_gather((b - 1) % NBUF)
            copy_sidx((b - 1) % NBUF)
            issue_scatter((b - 1) % NBUF)
        for i in (NCHUNKS - 2, NCHUNKS - 1):
            b = i % NBUF
            drain_scatter(b)
            drain_idx(i, b)
            issue_gather(b)
            drain_gather((b - 1) % NBUF)
            copy_sidx((b - 1) % NBUF)
            issue_scatter((b - 1) % NBUF)
        bl = (NCHUNKS - 1) % NBUF
        drain_gather(bl)
        copy_sidx(bl)
        issue_scatter(bl)
        for i in range(NCHUNKS - 4, NCHUNKS):
            drain_scatter(i % NBUF)
        plsc.subcore_barrier()

        pltpu.sync_copy(
            u_sh.at[pl.ds(sub * RPS, RPS)],
            up_out.at[core, pl.ds(sub * RPS, RPS)],
        )

        @pl.when(sub < NS - 1)
        def _pub_main():
            pltpu.sync_copy(
                acc_sh.at[pl.ds(sub * RPS, RPS)],
                p_out.at[core, pl.ds(sub * RPS, RPS)],
            )

        @pl.when(sub == NS - 1)
        def _pub_last():
            pltpu.sync_copy(
                acc_sh.at[pl.ds((NS - 1) * RPS, RPS_LAST)],
                p_out.at[core, pl.ds((NS - 1) * RPS, RPS_LAST)],
            )

    return k(src, dst, x, w3pad)


def _tc_tail(p, up, W1, b1, W2, b2, w3pad, b3):
    def body(p_ref, up_ref, w1_ref, b1_ref, w2_ref, b2_ref, w3_ref, b3_ref, o_ref):
        agg = p_ref[0] + p_ref[1]
        h1 = jnp.dot(agg, w1_ref[...], preferred_element_type=jnp.float32)
        h1 = jnp.maximum(h1 + b1_ref[...][None, :], 0.0)
        u = (up_ref[0] + up_ref[1])[:N]
        v = jnp.dot(u[None, :], h1, preferred_element_type=jnp.float32)
        s = jnp.sum(w3_ref[...])
        logits = jnp.dot(v, w2_ref[...], preferred_element_type=jnp.float32)
        logits = logits + s * b2_ref[...][None, :] + b3_ref[...][None, :]
        o_ref[...] = (1.0 / (1.0 + jnp.exp(-logits))).reshape(C, 1)

    return pl.pallas_call(
        body,
        out_shape=jax.ShapeDtypeStruct((C, 1), jnp.float32),
    )(p, up, W1, b1, W2, b2, w3pad, b3)


@jax.jit
def kernel(inputs, edge_index, W1, b1, W2, b2, W3, b3):
    src = edge_index[0].reshape(E // CHUNK, CHUNK)
    dst = edge_index[1].reshape(E // CHUNK, CHUNK)
    w3flat = W3[:, 0]
    p, up = _sc_edge_pass(src, dst, inputs, w3flat)
    return _tc_tail(p, up, W1, b1, W2, b2, w3flat, b3)
